# baseline (speedup 1.0000x reference)
import functools

import jax
import jax.numpy as jnp
from jax import lax
from jax.experimental import pallas as pl
from jax.experimental.pallas import tpu as pltpu
from jax.experimental.pallas import tpu_sc as plsc

N = 10000
E = 320000
G = 100
K = 30

_NPr = 10112
_TPT = _NPr // 16
_NCHUNK = 80
_EPAD = 32 * _NCHUNK * 128
_NPAD2 = 10240

@functools.lru_cache(maxsize=None)
def _make_edge_agg(W):
    @functools.partial(
        pl.kernel,
        out_type=jax.ShapeDtypeStruct((2, _NPr, W), jnp.float32),
        mesh=plsc.VectorSubcoreMesh(core_axis_name="c", subcore_axis_name="s"),
        compiler_params=pltpu.CompilerParams(use_tc_tiling_on_sc=False),
        scratch_types=[
            pltpu.VMEM((_NCHUNK, 128), jnp.int32),
            pltpu.VMEM((_NCHUNK, 128), jnp.int32),
            pltpu.VMEM((128, W), jnp.float32),
            pltpu.VMEM((128, W), jnp.float32),
            pltpu.VMEM((_TPT, W), jnp.float32),
            pltpu.VMEM_SHARED((_NPr, W), jnp.float32),
            pltpu.SemaphoreType.DMA,
            pltpu.SemaphoreType.DMA,
        ],
    )
    def k(table_hbm, srci_hbm, dsti_hbm, zeros_hbm, out_hbm,
          sidx, didx, rows0, rows1, tbuf, acc, sem0, sem1):
        c = lax.axis_index("c")
        s = lax.axis_index("s")
        wid = c * 16 + s
        rows = (rows0, rows1)
        sems = (sem0, sem1)
        pltpu.sync_copy(srci_hbm.at[wid], sidx)
        pltpu.sync_copy(dsti_hbm.at[wid], didx)
        pltpu.sync_copy(zeros_hbm, tbuf)
        pltpu.sync_copy(tbuf, acc.at[pl.ds(s * _TPT, _TPT)])
        plsc.subcore_barrier()
        pltpu.async_copy(table_hbm.at[sidx.at[0]], rows0, sem0)
        pltpu.async_copy(table_hbm.at[sidx.at[1]], rows1, sem1)
        for j in range(_NCHUNK):
            b = j & 1
            pltpu.make_async_copy(table_hbm.at[sidx.at[j]], rows[b],
                                  sems[b]).wait()
            pltpu.sync_copy(rows[b], acc.at[didx.at[j]], add=True)
            if j + 2 < _NCHUNK:
                pltpu.async_copy(table_hbm.at[sidx.at[j + 2]], rows[b],
                                 sems[b])
        plsc.subcore_barrier()
        pltpu.sync_copy(acc.at[pl.ds(s * _TPT, _TPT)], tbuf)
        pltpu.sync_copy(tbuf, out_hbm.at[c, pl.ds(s * _TPT, _TPT)])

    return k


def _sc_agg32(*args):
    return _make_edge_agg(32)(*args)


def _sc_agg16(*args):
    return _make_edge_agg(16)(*args)


@functools.lru_cache(maxsize=None)
def _make_deg():
    @functools.partial(
        pl.kernel,
        out_type=jax.ShapeDtypeStruct((2, _NPr, 16), jnp.float32),
        mesh=plsc.VectorSubcoreMesh(core_axis_name="c", subcore_axis_name="s"),
        compiler_params=pltpu.CompilerParams(use_tc_tiling_on_sc=False),
        scratch_types=[
            pltpu.VMEM((_NCHUNK, 128), jnp.int32),
            pltpu.VMEM((128, 16), jnp.float32),
            pltpu.VMEM((_TPT, 16), jnp.float32),
            pltpu.VMEM_SHARED((_NPr, 16), jnp.float32),
        ],
    )
    def k(ones_hbm, dsti_hbm, zeros_hbm, out_hbm, didx, vals, tbuf, acc):
        c = lax.axis_index("c")
        s = lax.axis_index("s")
        wid = c * 16 + s
        pltpu.sync_copy(dsti_hbm.at[wid], didx)
        pltpu.sync_copy(ones_hbm, vals)
        pltpu.sync_copy(zeros_hbm, tbuf)
        pltpu.sync_copy(tbuf, acc.at[pl.ds(s * _TPT, _TPT)])
        plsc.subcore_barrier()
        for j in range(_NCHUNK):
            pltpu.sync_copy(vals, acc.at[didx.at[j]], add=True)
        plsc.subcore_barrier()
        pltpu.sync_copy(acc.at[pl.ds(s * _TPT, _TPT)], tbuf)
        pltpu.sync_copy(tbuf, out_hbm.at[c, pl.ds(s * _TPT, _TPT)])

    return k


def _sc_deg(*args):
    return _make_deg()(*args)


@functools.lru_cache(maxsize=None)
def _make_rowgather():
    @functools.partial(
        pl.kernel,
        out_type=jax.ShapeDtypeStruct((3072, 128), jnp.float32),
        mesh=plsc.VectorSubcoreMesh(core_axis_name="c", subcore_axis_name="s"),
        compiler_params=pltpu.CompilerParams(use_tc_tiling_on_sc=False),
        scratch_types=[
            pltpu.VMEM((96,), jnp.int32),
            pltpu.VMEM((96, 128), jnp.float32),
            pltpu.SemaphoreType.DMA,
        ],
    )
    def k(table_hbm, idx_hbm, out_hbm, idxv, rows, sem):
        c = lax.axis_index("c")
        s = lax.axis_index("s")
        wid = c * 16 + s
        pltpu.sync_copy(idx_hbm.at[wid], idxv)
        pltpu.async_copy(table_hbm.at[idxv], rows, sem).wait()
        pltpu.sync_copy(rows, out_hbm.at[pl.ds(wid * 96, 96)])

    return k


def _sc_rowgather(*args):
    return _make_rowgather()(*args)


def _prep_body(s_ref, d_ref, o_ref):
    s = s_ref[...]
    d = d_ref[...]
    e = (lax.broadcasted_iota(jnp.int32, s.shape, 0) * 128
         + lax.broadcasted_iota(jnp.int32, s.shape, 1))
    o_ref[...] = jnp.where(s == d, N + (e & 15), d)


def _layer0_body(x_ref, w_ref, degp_ref, hs_ref, dis_ref):
    degp = degp_ref[...]
    deg = degp[0, :N, 0:1] + degp[1, :N, 0:1]
    dis = lax.rsqrt(deg + 1.0)
    h = jnp.dot(x_ref[...], w_ref[...], preferred_element_type=jnp.float32)
    hs_ref[...] = dis * h
    dis_ref[...] = dis


def _combine_body(p_ref, hs_ref, dis_ref, b_ref, wn_ref, x_ref, hsn_ref):
    p = p_ref[...]
    s = p[0, :N, :] + p[1, :N, :]
    dis = dis_ref[...]
    xl = jnp.tanh(dis * (s + hs_ref[...]) + b_ref[...])
    x_ref[...] = xl
    hsn_ref[...] = dis * jnp.dot(xl, wn_ref[...],
                                 preferred_element_type=jnp.float32)


def _last_body(p_ref, hs_ref, dis_ref, b_ref, v_ref):
    p = p_ref[...]
    s = p[0, :N, :] + p[1, :N, :]
    t = jnp.tanh(dis_ref[...] * (s + hs_ref[...]) + b_ref[...])
    v_ref[...] = t[:, 0:1]


def _topk_body(v_ref, b_ref, o_ref, work_ref):
    g = pl.program_id(0)
    v = v_ref[...]
    b = b_ref[...]
    neg = jnp.float32(-jnp.inf)
    work = jnp.where(b == g, v, neg)
    work_ref[...] = work
    rm = jnp.max(work, axis=1, keepdims=True)
    rowi = lax.broadcasted_iota(jnp.int32, (80, 1), 0)
    lanei = lax.broadcasted_iota(jnp.int32, (1, 128), 1)
    lane32 = lax.broadcasted_iota(jnp.int32, (1, 32), 1)
    big = jnp.int32(2 ** 30)
    out = jnp.zeros((1, 32), jnp.int32)
    for j in range(K):
        m = jnp.max(rm)
        valid = m > neg
        r = jnp.min(jnp.where(rm == m, rowi, big))
        row = work_ref[pl.ds(r, 1), :]
        li = jnp.min(jnp.where(row == m, lanei, big))
        sel = jnp.where(valid, r * 128 + li, N + ((g * K + j) & 7))
        out = jnp.where(lane32 == j, sel, out)
        row = jnp.where(lanei == li, neg, row)
        work_ref[pl.ds(r, 1), :] = row
        rm = jnp.where(rowi == r, jnp.max(row), rm)
    o_ref[...] = out.reshape(1, 1, 32)


def _head1_body(g_ref, w_ref, b_ref, o_ref):
    o_ref[...] = jax.nn.relu(
        jnp.dot(g_ref[...], w_ref[...], preferred_element_type=jnp.float32)
        + b_ref[...])


def _head2_body(a_ref, b2_ref, w_ref, bb_ref, o_ref):
    z = jnp.maximum(a_ref[...], b2_ref[...])
    o_ref[...] = jax.nn.relu(
        jnp.dot(z, w_ref[...], preferred_element_type=jnp.float32)
        + bb_ref[...])


def _head3_body(y_ref, fw_ref, fb_ref, gw_ref, gb_ref, o_ref):
    h = jax.nn.relu(
        jnp.dot(y_ref[...], fw_ref[...], preferred_element_type=jnp.float32)
        + fb_ref[...])
    logits = jnp.dot(h, gw_ref[...],
                     preferred_element_type=jnp.float32) + gb_ref[...]
    m = jnp.max(logits, axis=1, keepdims=True)
    lse = m + jnp.log(jnp.sum(jnp.exp(logits - m), axis=1, keepdims=True))
    o_ref[...] = logits - lse


def _sds(shape):
    return jax.ShapeDtypeStruct(shape, jnp.float32)


def kernel(x, edge_index, batch, W1, b1, W2, b2, W3, b3, W4, b4, conv5_w,
           conv5_b, conv6_w, conv6_b, fc1_w, fc1_b, fc2_w, fc2_b):
    f32 = jnp.float32
    i32 = jnp.int32
    src = edge_index[0]
    dst = edge_index[1]

    dste2d = pl.pallas_call(
        _prep_body,
        out_shape=jax.ShapeDtypeStruct((2500, 128), i32),
    )(src.reshape(2500, 128), dst.reshape(2500, 128))
    npad = _EPAD - E
    pad_dst = N + (jnp.arange(npad, dtype=i32) & 15)
    pad_src = (jnp.arange(npad, dtype=i32) * 97) % N
    dsti = jnp.concatenate([dste2d.reshape(E), pad_dst]).reshape(32, _NCHUNK, 128)
    srci = jnp.concatenate([src, pad_src]).reshape(32, _NCHUNK, 128)
    z32 = jnp.zeros((_TPT, 32), f32)
    z16 = jnp.zeros((_TPT, 16), f32)

    ones_t = jnp.concatenate([jnp.ones((128, 1), f32), jnp.zeros((128, 15), f32)], 1)
    degp = _sc_deg(ones_t, dsti, z16)

    hs1, dis = pl.pallas_call(
        _layer0_body,
        out_shape=[_sds((N, 32)), _sds((N, 1))],
    )(x, W1, degp)

    def combine(P, hs, b2d, Wn, wout):
        return pl.pallas_call(
            _combine_body,
            out_shape=[_sds((N, 32)), _sds((N, wout))],
        )(P, hs, dis, b2d, Wn)

    P1 = _sc_agg32(hs1, srci, dsti, z32)
    x1, hs2 = combine(P1, hs1, b1.reshape(1, 32), W2, 32)
    P2 = _sc_agg32(hs2, srci, dsti, z32)
    x2, hs3 = combine(P2, hs2, b2.reshape(1, 32), W3, 32)
    P3 = _sc_agg32(hs3, srci, dsti, z32)
    W4p = jnp.pad(W4, ((0, 0), (0, 15)))
    x3, hs4 = combine(P3, hs3, b3.reshape(1, 32), W4p, 16)
    P4 = _sc_agg16(hs4, srci, dsti, z16)
    b4p = jnp.pad(b4.reshape(1, 1), ((0, 0), (0, 15)))
    vcol = pl.pallas_call(
        _last_body,
        out_shape=_sds((N, 1)),
    )(P4, hs4, dis, b4p)

    v2d = jnp.pad(vcol.reshape(N), (0, _NPAD2 - N),
                  constant_values=-jnp.inf).reshape(80, 128)
    batch2d = jnp.pad(batch, (0, _NPAD2 - N),
                      constant_values=-1).reshape(80, 128)
    idx3 = pl.pallas_call(
        _topk_body,
        grid=(G,),
        in_specs=[
            pl.BlockSpec((80, 128), lambda g: (0, 0)),
            pl.BlockSpec((80, 128), lambda g: (0, 0)),
        ],
        out_specs=pl.BlockSpec((1, 1, 32), lambda g: (g, 0, 0)),
        out_shape=jax.ShapeDtypeStruct((G, 1, 32), i32),
        scratch_shapes=[pltpu.VMEM((80, 128), f32)],
    )(v2d, batch2d)
    idxs = idx3[:, 0, :K].reshape(G * K)
    pad_g = N + (jnp.arange(72, dtype=i32) & 7)
    idx_g = jnp.concatenate([idxs, pad_g]).reshape(32, 96)

    xc = jnp.concatenate([x1, x2, x3, vcol, jnp.zeros((N, 31), f32)], 1)
    xc_ext = jnp.pad(xc, ((0, 16), (0, 0)))
    rows = _sc_rowgather(xc_ext, idx_g)

    W5p = jnp.pad(conv5_w[:, 0, :].T, ((0, 31), (0, 0)))
    Y5 = pl.pallas_call(
        _head1_body,
        out_shape=_sds((3072, 16)),
    )(rows, W5p, conv5_b.reshape(1, 16))

    Y5r = Y5[:G * K].reshape(G, K, 16)
    Y5e = Y5r[:, 0::2, :]
    Y5o = Y5r[:, 1::2, :]
    A2 = jnp.stack([Y5e[:, t:t + 5, :] for t in range(11)], 1).reshape(1100, 80)
    B2 = jnp.stack([Y5o[:, t:t + 5, :] for t in range(11)], 1).reshape(1100, 80)
    W6m = conv6_w.transpose(2, 1, 0).reshape(80, 32)
    Y6 = pl.pallas_call(
        _head2_body,
        out_shape=_sds((1100, 32)),
    )(A2, B2, W6m, conv6_b.reshape(1, 32))

    fc1_wr = fc1_w.reshape(32, 11, 128).transpose(1, 0, 2).reshape(352, 128)
    out = pl.pallas_call(
        _head3_body,
        out_shape=_sds((G, 10)),
    )(Y6.reshape(G, 352), fc1_wr, fc1_b.reshape(1, 128),
      fc2_w, fc2_b.reshape(1, 10))
    return out


# speedup vs baseline: 3.3554x; 3.3554x over previous
import functools

import jax
import jax.numpy as jnp
from jax import lax
from jax.experimental import pallas as pl
from jax.experimental.pallas import tpu as pltpu
from jax.experimental.pallas import tpu_sc as plsc

N = 10000
E = 320000
G = 100
K = 30

_NPr = 10112
_TPT = _NPr // 16
_NCHUNK = 80
_EPAD = 32 * _NCHUNK * 128
_NPAD2 = 10240

@functools.lru_cache(maxsize=None)
def _make_edge_agg(W, npr, nchunk):
    tpt = npr // 16

    @functools.partial(
        pl.kernel,
        out_type=jax.ShapeDtypeStruct((2, npr, W), jnp.float32),
        mesh=plsc.VectorSubcoreMesh(core_axis_name="c", subcore_axis_name="s"),
        compiler_params=pltpu.CompilerParams(use_tc_tiling_on_sc=False),
        scratch_types=[
            pltpu.VMEM((nchunk, 128), jnp.int32),
            pltpu.VMEM((nchunk, 128), jnp.int32),
            pltpu.VMEM((128, W), jnp.float32),
            pltpu.VMEM((128, W), jnp.float32),
            pltpu.VMEM((tpt, W), jnp.float32),
            pltpu.VMEM_SHARED((npr, W), jnp.float32),
            pltpu.SemaphoreType.DMA,
            pltpu.SemaphoreType.DMA,
        ],
    )
    def k(table_hbm, srci_hbm, dsti_hbm, zeros_hbm, out_hbm,
          sidx, didx, rows0, rows1, tbuf, acc, sem0, sem1):
        c = lax.axis_index("c")
        s = lax.axis_index("s")
        wid = c * 16 + s
        rows = (rows0, rows1)
        sems = (sem0, sem1)
        pltpu.sync_copy(srci_hbm.at[wid], sidx)
        pltpu.sync_copy(dsti_hbm.at[wid], didx)
        pltpu.sync_copy(zeros_hbm, tbuf)
        pltpu.sync_copy(tbuf, acc.at[pl.ds(s * tpt, tpt)])
        plsc.subcore_barrier()
        pltpu.async_copy(table_hbm.at[sidx.at[0]], rows0, sem0)
        pltpu.async_copy(table_hbm.at[sidx.at[1]], rows1, sem1)
        for j in range(nchunk):
            b = j & 1
            pltpu.make_async_copy(table_hbm.at[sidx.at[j]], rows[b],
                                  sems[b]).wait()
            pltpu.sync_copy(rows[b], acc.at[didx.at[j]], add=True)
            if j + 2 < nchunk:
                pltpu.async_copy(table_hbm.at[sidx.at[j + 2]], rows[b],
                                 sems[b])
        plsc.subcore_barrier()
        pltpu.sync_copy(acc.at[pl.ds(s * tpt, tpt)], tbuf)
        pltpu.sync_copy(tbuf, out_hbm.at[c, pl.ds(s * tpt, tpt)])

    return k


def _sc_agg32(*args):
    return _make_edge_agg(32, _NPr, _NCHUNK)(*args)


def _sc_agg16(*args):
    return _make_edge_agg(16, _NPr, _NCHUNK)(*args)


_NSLOT = 3328


def _sc_slotscatter(*args):
    return _make_edge_agg(128, _NSLOT, 3)(*args)


@functools.lru_cache(maxsize=None)
def _make_deg():
    @functools.partial(
        pl.kernel,
        out_type=jax.ShapeDtypeStruct((2, _NPr, 16), jnp.float32),
        mesh=plsc.VectorSubcoreMesh(core_axis_name="c", subcore_axis_name="s"),
        compiler_params=pltpu.CompilerParams(use_tc_tiling_on_sc=False),
        scratch_types=[
            pltpu.VMEM((_NCHUNK, 128), jnp.int32),
            pltpu.VMEM((128, 16), jnp.float32),
            pltpu.VMEM((_TPT, 16), jnp.float32),
            pltpu.VMEM_SHARED((_NPr, 16), jnp.float32),
        ],
    )
    def k(ones_hbm, dsti_hbm, zeros_hbm, out_hbm, didx, vals, tbuf, acc):
        c = lax.axis_index("c")
        s = lax.axis_index("s")
        wid = c * 16 + s
        pltpu.sync_copy(dsti_hbm.at[wid], didx)
        pltpu.sync_copy(ones_hbm, vals)
        pltpu.sync_copy(zeros_hbm, tbuf)
        pltpu.sync_copy(tbuf, acc.at[pl.ds(s * _TPT, _TPT)])
        plsc.subcore_barrier()
        for j in range(_NCHUNK):
            pltpu.sync_copy(vals, acc.at[didx.at[j]], add=True)
        plsc.subcore_barrier()
        pltpu.sync_copy(acc.at[pl.ds(s * _TPT, _TPT)], tbuf)
        pltpu.sync_copy(tbuf, out_hbm.at[c, pl.ds(s * _TPT, _TPT)])

    return k


def _sc_deg(*args):
    return _make_deg()(*args)


def _prep_body(s_ref, d_ref, o_ref):
    s = s_ref[...]
    d = d_ref[...]
    e = (lax.broadcasted_iota(jnp.int32, s.shape, 0) * 128
         + lax.broadcasted_iota(jnp.int32, s.shape, 1))
    o_ref[...] = jnp.where(s == d, N + (e & 15), d)


def _layer0_body(x_ref, w_ref, degp_ref, hs_ref, dis_ref):
    degp = degp_ref[...]
    deg = degp[0, :N, 0:1] + degp[1, :N, 0:1]
    dis = lax.rsqrt(deg + 1.0)
    h = jnp.dot(x_ref[...], w_ref[...], preferred_element_type=jnp.float32)
    hs_ref[...] = dis * h
    dis_ref[...] = dis


def _combine_body(p_ref, hs_ref, dis_ref, b_ref, wn_ref, x_ref, hsn_ref):
    p = p_ref[...]
    s = p[0, :N, :] + p[1, :N, :]
    dis = dis_ref[...]
    xl = jnp.tanh(dis * (s + hs_ref[...]) + b_ref[...])
    x_ref[...] = xl
    hsn_ref[...] = dis * jnp.dot(xl, wn_ref[...],
                                 preferred_element_type=jnp.float32)


def _last_body(p_ref, hs_ref, dis_ref, b_ref, v_ref):
    p = p_ref[...]
    s = p[0, :N, :] + p[1, :N, :]
    t = jnp.tanh(dis_ref[...] * (s + hs_ref[...]) + b_ref[...])
    v_ref[...] = t[:, 0:1]


def _rank_body(v2d_ref, b2d_ref, vs_ref, bs_ref, o_ref):
    big = jnp.int32(2 ** 30)
    allb = b2d_ref[...]
    allflat = (lax.broadcasted_iota(jnp.int32, (80, 128), 0) * 128
               + lax.broadcasted_iota(jnp.int32, (80, 128), 1))
    for blk in range(10):
        vi = v2d_ref[8 * blk:8 * blk + 8, :]
        bi = b2d_ref[8 * blk:8 * blk + 8, :]
        flati = (lax.broadcasted_iota(jnp.int32, (8, 128), 0) * 128
                 + lax.broadcasted_iota(jnp.int32, (8, 128), 1)
                 + 1024 * blk)
        bfirst = bs_ref[8 * blk * 128]
        blast = bs_ref[8 * blk * 128 + 1023]
        span_lo = jnp.min(jnp.where(allb == bfirst, allflat, big))
        span_hi = jnp.max(jnp.where(allb == blast, allflat, -1)) + 1
        lo8 = (span_lo // 8) * 8
        nsteps = (span_hi - lo8 + 7) // 8

        def jbody(t, cnt):
            base = lo8 + t * 8
            for u in range(8):
                j = base + u
                vj = vs_ref[j]
                bj = bs_ref[j]
                beats = (vj > vi) | ((vj == vi) & (j < flati))
                cnt = cnt + jnp.where(beats & (bj == bi), 1, 0)
            return cnt

        cnt = lax.fori_loop(0, nsteps, jbody,
                            jnp.zeros((8, 128), jnp.int32))
        slot = jnp.where((cnt < K) & (bi >= 0), bi * 32 + cnt,
                         3200 + (flati & 127))
        o_ref[8 * blk:8 * blk + 8, :] = slot


def _head1_body(p_ref, w_ref, b_ref, o_ref):
    p = p_ref[...]
    rows = p[0] + p[1]
    o_ref[...] = jax.nn.relu(
        jnp.dot(rows, w_ref[...], preferred_element_type=jnp.float32)
        + b_ref[...])


def _head2_body(a_ref, b2_ref, w_ref, bb_ref, o_ref):
    z = jnp.maximum(a_ref[...], b2_ref[...])
    o_ref[...] = jax.nn.relu(
        jnp.dot(z, w_ref[...], preferred_element_type=jnp.float32)
        + bb_ref[...])


def _head3_body(y_ref, fw_ref, fb_ref, gw_ref, gb_ref, o_ref):
    h = jax.nn.relu(
        jnp.dot(y_ref[...], fw_ref[...], preferred_element_type=jnp.float32)
        + fb_ref[...])
    logits = jnp.dot(h, gw_ref[...],
                     preferred_element_type=jnp.float32) + gb_ref[...]
    m = jnp.max(logits, axis=1, keepdims=True)
    lse = m + jnp.log(jnp.sum(jnp.exp(logits - m), axis=1, keepdims=True))
    o_ref[...] = logits - lse


def _sds(shape):
    return jax.ShapeDtypeStruct(shape, jnp.float32)


def kernel(x, edge_index, batch, W1, b1, W2, b2, W3, b3, W4, b4, conv5_w,
           conv5_b, conv6_w, conv6_b, fc1_w, fc1_b, fc2_w, fc2_b):
    f32 = jnp.float32
    i32 = jnp.int32
    src = edge_index[0]
    dst = edge_index[1]

    dste2d = pl.pallas_call(
        _prep_body,
        out_shape=jax.ShapeDtypeStruct((2500, 128), i32),
    )(src.reshape(2500, 128), dst.reshape(2500, 128))
    npad = _EPAD - E
    pad_dst = N + (jnp.arange(npad, dtype=i32) & 15)
    pad_src = (jnp.arange(npad, dtype=i32) * 97) % N
    dsti = jnp.concatenate([dste2d.reshape(E), pad_dst]).reshape(32, _NCHUNK, 128)
    srci = jnp.concatenate([src, pad_src]).reshape(32, _NCHUNK, 128)
    z32 = jnp.zeros((_TPT, 32), f32)
    z16 = jnp.zeros((_TPT, 16), f32)

    ones_t = jnp.concatenate([jnp.ones((128, 1), f32), jnp.zeros((128, 15), f32)], 1)
    degp = _sc_deg(ones_t, dsti, z16)

    hs1, dis = pl.pallas_call(
        _layer0_body,
        out_shape=[_sds((N, 32)), _sds((N, 1))],
    )(x, W1, degp)

    def combine(P, hs, b2d, Wn, wout):
        return pl.pallas_call(
            _combine_body,
            out_shape=[_sds((N, 32)), _sds((N, wout))],
        )(P, hs, dis, b2d, Wn)

    P1 = _sc_agg32(hs1, srci, dsti, z32)
    x1, hs2 = combine(P1, hs1, b1.reshape(1, 32), W2, 32)
    P2 = _sc_agg32(hs2, srci, dsti, z32)
    x2, hs3 = combine(P2, hs2, b2.reshape(1, 32), W3, 32)
    P3 = _sc_agg32(hs3, srci, dsti, z32)
    W4p = jnp.pad(W4, ((0, 0), (0, 15)))
    x3, hs4 = combine(P3, hs3, b3.reshape(1, 32), W4p, 16)
    P4 = _sc_agg16(hs4, srci, dsti, z16)
    b4p = jnp.pad(b4.reshape(1, 1), ((0, 0), (0, 15)))
    vcol = pl.pallas_call(
        _last_body,
        out_shape=_sds((N, 1)),
    )(P4, hs4, dis, b4p)

    v2d = jnp.pad(vcol.reshape(N), (0, _NPAD2 - N),
                  constant_values=-jnp.inf).reshape(80, 128)
    batch2d = jnp.pad(batch, (0, _NPAD2 - N),
                      constant_values=-1).reshape(80, 128)
    slot2d = pl.pallas_call(
        _rank_body,
        in_specs=[
            pl.BlockSpec(memory_space=pltpu.VMEM),
            pl.BlockSpec(memory_space=pltpu.VMEM),
            pl.BlockSpec(memory_space=pltpu.SMEM),
            pl.BlockSpec(memory_space=pltpu.SMEM),
        ],
        out_shape=jax.ShapeDtypeStruct((80, 128), i32),
    )(v2d, batch2d, v2d.reshape(_NPAD2), batch2d.reshape(_NPAD2))

    xc = jnp.concatenate([x1, x2, x3, vcol, jnp.zeros((N, 31), f32)], 1)
    xc_ext = jnp.pad(xc, ((0, _NPAD2 - N), (0, 0)))
    npad_n = 32 * 3 * 128 - _NPAD2
    srcn = jnp.concatenate([jnp.arange(_NPAD2, dtype=i32),
                            (jnp.arange(npad_n, dtype=i32) * 97) % N])
    dstn = jnp.concatenate([slot2d.reshape(_NPAD2),
                            3200 + (jnp.arange(npad_n, dtype=i32) & 127)])
    P5 = _sc_slotscatter(xc_ext, srcn.reshape(32, 3, 128),
                         dstn.reshape(32, 3, 128),
                         jnp.zeros((_NSLOT // 16, 128), f32))

    W5p = jnp.pad(conv5_w[:, 0, :].T, ((0, 31), (0, 0)))
    Y5 = pl.pallas_call(
        _head1_body,
        out_shape=_sds((_NSLOT, 16)),
    )(P5, W5p, conv5_b.reshape(1, 16))

    Y5r = Y5[:G * 32].reshape(G, 32, 16)[:, :K, :]
    Y5e = Y5r[:, 0::2, :]
    Y5o = Y5r[:, 1::2, :]
    A2 = jnp.stack([Y5e[:, t:t + 5, :] for t in range(11)], 1).reshape(1100, 80)
    B2 = jnp.stack([Y5o[:, t:t + 5, :] for t in range(11)], 1).reshape(1100, 80)
    W6m = conv6_w.transpose(2, 1, 0).reshape(80, 32)
    Y6 = pl.pallas_call(
        _head2_body,
        out_shape=_sds((1100, 32)),
    )(A2, B2, W6m, conv6_b.reshape(1, 32))

    fc1_wr = fc1_w.reshape(32, 11, 128).transpose(1, 0, 2).reshape(352, 128)
    out = pl.pallas_call(
        _head3_body,
        out_shape=_sds((G, 10)),
    )(Y6.reshape(G, 352), fc1_wr, fc1_b.reshape(1, 128),
      fc2_w, fc2_b.reshape(1, 10))
    return out


# speedup vs baseline: 3.5213x; 1.0494x over previous
import functools

import jax
import jax.numpy as jnp
from jax import lax
from jax.experimental import pallas as pl
from jax.experimental.pallas import tpu as pltpu
from jax.experimental.pallas import tpu_sc as plsc

N = 10000
E = 320000
G = 100
K = 30

_NPr = 10112
_TPT = _NPr // 16
_NCHUNK = 80
_EPAD = 32 * _NCHUNK * 128
_NPAD2 = 10240

@functools.lru_cache(maxsize=None)
def _make_edge_agg(W, npr, nchunk):
    tpt = npr // 16

    @functools.partial(
        pl.kernel,
        out_type=jax.ShapeDtypeStruct((2, npr, W), jnp.float32),
        mesh=plsc.VectorSubcoreMesh(core_axis_name="c", subcore_axis_name="s"),
        compiler_params=pltpu.CompilerParams(use_tc_tiling_on_sc=False),
        scratch_types=[
            pltpu.VMEM((nchunk, 128), jnp.int32),
            pltpu.VMEM((nchunk, 128), jnp.int32),
            pltpu.VMEM((128, W), jnp.float32),
            pltpu.VMEM((128, W), jnp.float32),
            pltpu.VMEM((128, W), jnp.float32),
            pltpu.VMEM((128, W), jnp.float32),
            pltpu.VMEM((tpt, W), jnp.float32),
            pltpu.VMEM_SHARED((npr, W), jnp.float32),
            pltpu.SemaphoreType.DMA,
            pltpu.SemaphoreType.DMA,
            pltpu.SemaphoreType.DMA,
            pltpu.SemaphoreType.DMA,
            pltpu.SemaphoreType.DMA,
            pltpu.SemaphoreType.DMA,
            pltpu.SemaphoreType.DMA,
            pltpu.SemaphoreType.DMA,
        ],
    )
    def k(table_hbm, srci_hbm, dsti_hbm, zeros_hbm, out_hbm,
          sidx, didx, r0, r1, r2, r3, tbuf, acc,
          g0, g1, g2, g3, s0, s1, s2, s3):
        c = lax.axis_index("c")
        s = lax.axis_index("s")
        wid = c * 16 + s
        rows = (r0, r1, r2, r3)
        gsem = (g0, g1, g2, g3)
        ssem = (s0, s1, s2, s3)
        pltpu.sync_copy(srci_hbm.at[wid], sidx)
        pltpu.sync_copy(dsti_hbm.at[wid], didx)
        pltpu.sync_copy(zeros_hbm, tbuf)
        pltpu.sync_copy(tbuf, acc.at[pl.ds(s * tpt, tpt)])
        plsc.subcore_barrier()
        pltpu.async_copy(table_hbm.at[sidx.at[0]], rows[0], gsem[0])
        if nchunk > 1:
            pltpu.async_copy(table_hbm.at[sidx.at[1]], rows[1], gsem[1])
        for j in range(nchunk):
            b = j % 4
            pltpu.make_async_copy(table_hbm.at[sidx.at[j]], rows[b],
                                  gsem[b]).wait()
            pltpu.async_copy(rows[b], acc.at[didx.at[j]], ssem[b], add=True)
            if j + 2 < nchunk:
                b2 = (j + 2) % 4
                if j >= 2:
                    pltpu.make_async_copy(rows[b2], acc.at[didx.at[j - 2]],
                                          ssem[b2]).wait()
                pltpu.async_copy(table_hbm.at[sidx.at[j + 2]], rows[b2],
                                 gsem[b2])
        for j in range(max(nchunk - 4, 0), nchunk):
            b = j % 4
            pltpu.make_async_copy(rows[b], acc.at[didx.at[j]],
                                  ssem[b]).wait()
        plsc.subcore_barrier()
        pltpu.sync_copy(acc.at[pl.ds(s * tpt, tpt)], tbuf)
        pltpu.sync_copy(tbuf, out_hbm.at[c, pl.ds(s * tpt, tpt)])

    return k


def _sc_agg32(*args):
    return _make_edge_agg(32, _NPr, _NCHUNK)(*args)


def _sc_agg16(*args):
    return _make_edge_agg(16, _NPr, _NCHUNK)(*args)


_NSLOT = 3328


def _sc_slotscatter(*args):
    return _make_edge_agg(128, _NSLOT, 3)(*args)


@functools.lru_cache(maxsize=None)
def _make_deg():
    @functools.partial(
        pl.kernel,
        out_type=jax.ShapeDtypeStruct((2, _NPr, 16), jnp.float32),
        mesh=plsc.VectorSubcoreMesh(core_axis_name="c", subcore_axis_name="s"),
        compiler_params=pltpu.CompilerParams(use_tc_tiling_on_sc=False),
        scratch_types=[
            pltpu.VMEM((_NCHUNK, 128), jnp.int32),
            pltpu.VMEM((128, 16), jnp.float32),
            pltpu.VMEM((_TPT, 16), jnp.float32),
            pltpu.VMEM_SHARED((_NPr, 16), jnp.float32),
        ],
    )
    def k(ones_hbm, dsti_hbm, zeros_hbm, out_hbm, didx, vals, tbuf, acc):
        c = lax.axis_index("c")
        s = lax.axis_index("s")
        wid = c * 16 + s
        pltpu.sync_copy(dsti_hbm.at[wid], didx)
        pltpu.sync_copy(ones_hbm, vals)
        pltpu.sync_copy(zeros_hbm, tbuf)
        pltpu.sync_copy(tbuf, acc.at[pl.ds(s * _TPT, _TPT)])
        plsc.subcore_barrier()
        for j in range(_NCHUNK):
            pltpu.sync_copy(vals, acc.at[didx.at[j]], add=True)
        plsc.subcore_barrier()
        pltpu.sync_copy(acc.at[pl.ds(s * _TPT, _TPT)], tbuf)
        pltpu.sync_copy(tbuf, out_hbm.at[c, pl.ds(s * _TPT, _TPT)])

    return k


def _sc_deg(*args):
    return _make_deg()(*args)


def _prep_body(s_ref, d_ref, o_ref):
    s = s_ref[...]
    d = d_ref[...]
    e = (lax.broadcasted_iota(jnp.int32, s.shape, 0) * 128
         + lax.broadcasted_iota(jnp.int32, s.shape, 1))
    o_ref[...] = jnp.where(s == d, N + (e & 15), d)


def _layer0_body(x_ref, w_ref, degp_ref, hs_ref, dis_ref):
    degp = degp_ref[...]
    deg = degp[0, :N, 0:1] + degp[1, :N, 0:1]
    dis = lax.rsqrt(deg + 1.0)
    h = jnp.dot(x_ref[...], w_ref[...], preferred_element_type=jnp.float32)
    hs_ref[...] = dis * h
    dis_ref[...] = dis


def _combine_body(p_ref, hs_ref, dis_ref, b_ref, wn_ref, x_ref, hsn_ref):
    p = p_ref[...]
    s = p[0, :N, :] + p[1, :N, :]
    dis = dis_ref[...]
    xl = jnp.tanh(dis * (s + hs_ref[...]) + b_ref[...])
    x_ref[...] = xl
    hsn_ref[...] = dis * jnp.dot(xl, wn_ref[...],
                                 preferred_element_type=jnp.float32)


def _last_body(p_ref, hs_ref, dis_ref, b_ref, v_ref):
    p = p_ref[...]
    s = p[0, :N, :] + p[1, :N, :]
    t = jnp.tanh(dis_ref[...] * (s + hs_ref[...]) + b_ref[...])
    v_ref[...] = t[:, 0:1]


def _rank_body(v2d_ref, b2d_ref, vs_ref, bs_ref, o_ref):
    big = jnp.int32(2 ** 30)
    allb = b2d_ref[...]
    allflat = (lax.broadcasted_iota(jnp.int32, (80, 128), 0) * 128
               + lax.broadcasted_iota(jnp.int32, (80, 128), 1))
    for blk in range(10):
        vi = v2d_ref[8 * blk:8 * blk + 8, :]
        bi = b2d_ref[8 * blk:8 * blk + 8, :]
        flati = (lax.broadcasted_iota(jnp.int32, (8, 128), 0) * 128
                 + lax.broadcasted_iota(jnp.int32, (8, 128), 1)
                 + 1024 * blk)
        bfirst = bs_ref[8 * blk * 128]
        blast = bs_ref[8 * blk * 128 + 1023]
        span_lo = jnp.min(jnp.where(allb == bfirst, allflat, big))
        span_hi = jnp.max(jnp.where(allb == blast, allflat, -1)) + 1
        lo8 = (span_lo // 8) * 8
        nsteps = (span_hi - lo8 + 7) // 8

        def jbody(t, cnt):
            base = lo8 + t * 8
            for u in range(8):
                j = base + u
                vj = vs_ref[j]
                bj = bs_ref[j]
                beats = (vj > vi) | ((vj == vi) & (j < flati))
                cnt = cnt + jnp.where(beats & (bj == bi), 1, 0)
            return cnt

        cnt = lax.fori_loop(0, nsteps, jbody,
                            jnp.zeros((8, 128), jnp.int32))
        slot = jnp.where((cnt < K) & (bi >= 0), bi * 32 + cnt,
                         3200 + (flati & 127))
        o_ref[8 * blk:8 * blk + 8, :] = slot


def _head1_body(p_ref, w_ref, b_ref, o_ref):
    p = p_ref[...]
    rows = p[0] + p[1]
    o_ref[...] = jax.nn.relu(
        jnp.dot(rows, w_ref[...], preferred_element_type=jnp.float32)
        + b_ref[...])


def _head2_body(a_ref, b2_ref, w_ref, bb_ref, o_ref):
    z = jnp.maximum(a_ref[...], b2_ref[...])
    o_ref[...] = jax.nn.relu(
        jnp.dot(z, w_ref[...], preferred_element_type=jnp.float32)
        + bb_ref[...])


def _head3_body(y_ref, fw_ref, fb_ref, gw_ref, gb_ref, o_ref):
    h = jax.nn.relu(
        jnp.dot(y_ref[...], fw_ref[...], preferred_element_type=jnp.float32)
        + fb_ref[...])
    logits = jnp.dot(h, gw_ref[...],
                     preferred_element_type=jnp.float32) + gb_ref[...]
    m = jnp.max(logits, axis=1, keepdims=True)
    lse = m + jnp.log(jnp.sum(jnp.exp(logits - m), axis=1, keepdims=True))
    o_ref[...] = logits - lse


def _sds(shape):
    return jax.ShapeDtypeStruct(shape, jnp.float32)


def kernel(x, edge_index, batch, W1, b1, W2, b2, W3, b3, W4, b4, conv5_w,
           conv5_b, conv6_w, conv6_b, fc1_w, fc1_b, fc2_w, fc2_b):
    f32 = jnp.float32
    i32 = jnp.int32
    src = edge_index[0]
    dst = edge_index[1]

    dste2d = pl.pallas_call(
        _prep_body,
        out_shape=jax.ShapeDtypeStruct((2500, 128), i32),
    )(src.reshape(2500, 128), dst.reshape(2500, 128))
    npad = _EPAD - E
    pad_dst = N + (jnp.arange(npad, dtype=i32) & 15)
    pad_src = (jnp.arange(npad, dtype=i32) * 97) % N
    dsti = jnp.concatenate([dste2d.reshape(E), pad_dst]).reshape(32, _NCHUNK, 128)
    srci = jnp.concatenate([src, pad_src]).reshape(32, _NCHUNK, 128)
    z32 = jnp.zeros((_TPT, 32), f32)
    z16 = jnp.zeros((_TPT, 16), f32)

    ones_t = jnp.concatenate([jnp.ones((128, 1), f32), jnp.zeros((128, 15), f32)], 1)
    degp = _sc_deg(ones_t, dsti, z16)

    hs1, dis = pl.pallas_call(
        _layer0_body,
        out_shape=[_sds((N, 32)), _sds((N, 1))],
    )(x, W1, degp)

    def combine(P, hs, b2d, Wn, wout):
        return pl.pallas_call(
            _combine_body,
            out_shape=[_sds((N, 32)), _sds((N, wout))],
        )(P, hs, dis, b2d, Wn)

    P1 = _sc_agg32(hs1, srci, dsti, z32)
    x1, hs2 = combine(P1, hs1, b1.reshape(1, 32), W2, 32)
    P2 = _sc_agg32(hs2, srci, dsti, z32)
    x2, hs3 = combine(P2, hs2, b2.reshape(1, 32), W3, 32)
    P3 = _sc_agg32(hs3, srci, dsti, z32)
    W4p = jnp.pad(W4, ((0, 0), (0, 15)))
    x3, hs4 = combine(P3, hs3, b3.reshape(1, 32), W4p, 16)
    P4 = _sc_agg16(hs4, srci, dsti, z16)
    b4p = jnp.pad(b4.reshape(1, 1), ((0, 0), (0, 15)))
    vcol = pl.pallas_call(
        _last_body,
        out_shape=_sds((N, 1)),
    )(P4, hs4, dis, b4p)

    v2d = jnp.pad(vcol.reshape(N), (0, _NPAD2 - N),
                  constant_values=-jnp.inf).reshape(80, 128)
    batch2d = jnp.pad(batch, (0, _NPAD2 - N),
                      constant_values=-1).reshape(80, 128)
    slot2d = pl.pallas_call(
        _rank_body,
        in_specs=[
            pl.BlockSpec(memory_space=pltpu.VMEM),
            pl.BlockSpec(memory_space=pltpu.VMEM),
            pl.BlockSpec(memory_space=pltpu.SMEM),
            pl.BlockSpec(memory_space=pltpu.SMEM),
        ],
        out_shape=jax.ShapeDtypeStruct((80, 128), i32),
    )(v2d, batch2d, v2d.reshape(_NPAD2), batch2d.reshape(_NPAD2))

    xc = jnp.concatenate([x1, x2, x3, vcol, jnp.zeros((N, 31), f32)], 1)
    xc_ext = jnp.pad(xc, ((0, _NPAD2 - N), (0, 0)))
    npad_n = 32 * 3 * 128 - _NPAD2
    srcn = jnp.concatenate([jnp.arange(_NPAD2, dtype=i32),
                            (jnp.arange(npad_n, dtype=i32) * 97) % N])
    dstn = jnp.concatenate([slot2d.reshape(_NPAD2),
                            3200 + (jnp.arange(npad_n, dtype=i32) & 127)])
    P5 = _sc_slotscatter(xc_ext, srcn.reshape(32, 3, 128),
                         dstn.reshape(32, 3, 128),
                         jnp.zeros((_NSLOT // 16, 128), f32))

    W5p = jnp.pad(conv5_w[:, 0, :].T, ((0, 31), (0, 0)))
    Y5 = pl.pallas_call(
        _head1_body,
        out_shape=_sds((_NSLOT, 16)),
    )(P5, W5p, conv5_b.reshape(1, 16))

    Y5r = Y5[:G * 32].reshape(G, 32, 16)[:, :K, :]
    Y5e = Y5r[:, 0::2, :]
    Y5o = Y5r[:, 1::2, :]
    A2 = jnp.stack([Y5e[:, t:t + 5, :] for t in range(11)], 1).reshape(1100, 80)
    B2 = jnp.stack([Y5o[:, t:t + 5, :] for t in range(11)], 1).reshape(1100, 80)
    W6m = conv6_w.transpose(2, 1, 0).reshape(80, 32)
    Y6 = pl.pallas_call(
        _head2_body,
        out_shape=_sds((1100, 32)),
    )(A2, B2, W6m, conv6_b.reshape(1, 32))

    fc1_wr = fc1_w.reshape(32, 11, 128).transpose(1, 0, 2).reshape(352, 128)
    out = pl.pallas_call(
        _head3_body,
        out_shape=_sds((G, 10)),
    )(Y6.reshape(G, 352), fc1_wr, fc1_b.reshape(1, 128),
      fc2_w, fc2_b.reshape(1, 10))
    return out


# speedup vs baseline: 3.6979x; 1.0501x over previous
import functools

import jax
import jax.numpy as jnp
from jax import lax
from jax.experimental import pallas as pl
from jax.experimental.pallas import tpu as pltpu
from jax.experimental.pallas import tpu_sc as plsc

N = 10000
E = 320000
G = 100
K = 30

_NPr = 10112
_TPT = _NPr // 16
_NCHUNK = 80
_EPAD = 32 * _NCHUNK * 128
_NPAD2 = 10240

@functools.lru_cache(maxsize=None)
def _make_edge_agg(W, npr, nchunk):
    tpt = npr // 16

    @functools.partial(
        pl.kernel,
        out_type=jax.ShapeDtypeStruct((2, npr, W), jnp.float32),
        mesh=plsc.VectorSubcoreMesh(core_axis_name="c", subcore_axis_name="s"),
        compiler_params=pltpu.CompilerParams(use_tc_tiling_on_sc=False),
        scratch_types=[
            pltpu.VMEM((nchunk, 128), jnp.int32),
            pltpu.VMEM((nchunk, 128), jnp.int32),
            pltpu.VMEM((128, W), jnp.float32),
            pltpu.VMEM((128, W), jnp.float32),
            pltpu.VMEM((128, W), jnp.float32),
            pltpu.VMEM((128, W), jnp.float32),
            pltpu.VMEM((tpt, W), jnp.float32),
            pltpu.VMEM_SHARED((npr, W), jnp.float32),
            pltpu.SemaphoreType.DMA,
            pltpu.SemaphoreType.DMA,
            pltpu.SemaphoreType.DMA,
            pltpu.SemaphoreType.DMA,
            pltpu.SemaphoreType.DMA,
            pltpu.SemaphoreType.DMA,
            pltpu.SemaphoreType.DMA,
            pltpu.SemaphoreType.DMA,
        ],
    )
    def k(table_hbm, srci_hbm, dsti_hbm, zeros_hbm, out_hbm,
          sidx, didx, r0, r1, r2, r3, tbuf, acc,
          g0, g1, g2, g3, s0, s1, s2, s3):
        c = lax.axis_index("c")
        s = lax.axis_index("s")
        wid = c * 16 + s
        rows = (r0, r1, r2, r3)
        gsem = (g0, g1, g2, g3)
        ssem = (s0, s1, s2, s3)
        pltpu.sync_copy(srci_hbm.at[wid], sidx)
        pltpu.sync_copy(dsti_hbm.at[wid], didx)
        pltpu.sync_copy(zeros_hbm, tbuf)
        pltpu.sync_copy(tbuf, acc.at[pl.ds(s * tpt, tpt)])
        plsc.subcore_barrier()
        pltpu.async_copy(table_hbm.at[sidx.at[0]], rows[0], gsem[0])
        if nchunk > 1:
            pltpu.async_copy(table_hbm.at[sidx.at[1]], rows[1], gsem[1])
        for j in range(nchunk):
            b = j % 4
            pltpu.make_async_copy(table_hbm.at[sidx.at[j]], rows[b],
                                  gsem[b]).wait()
            pltpu.async_copy(rows[b], acc.at[didx.at[j]], ssem[b], add=True)
            if j + 2 < nchunk:
                b2 = (j + 2) % 4
                if j >= 2:
                    pltpu.make_async_copy(rows[b2], acc.at[didx.at[j - 2]],
                                          ssem[b2]).wait()
                pltpu.async_copy(table_hbm.at[sidx.at[j + 2]], rows[b2],
                                 gsem[b2])
        for j in range(max(nchunk - 4, 0), nchunk):
            b = j % 4
            pltpu.make_async_copy(rows[b], acc.at[didx.at[j]],
                                  ssem[b]).wait()
        plsc.subcore_barrier()
        pltpu.sync_copy(acc.at[pl.ds(s * tpt, tpt)], tbuf)
        pltpu.sync_copy(tbuf, out_hbm.at[c, pl.ds(s * tpt, tpt)])

    return k


def _sc_agg32(*args):
    return _make_edge_agg(32, _NPr, _NCHUNK)(*args)


def _sc_agg16(*args):
    return _make_edge_agg(16, _NPr, _NCHUNK)(*args)


_NSLOT = 3328


def _sc_slotscatter(*args):
    return _make_edge_agg(128, _NSLOT, 3)(*args)


@functools.lru_cache(maxsize=None)
def _make_deg():
    @functools.partial(
        pl.kernel,
        out_type=jax.ShapeDtypeStruct((2, _NPr, 16), jnp.float32),
        mesh=plsc.VectorSubcoreMesh(core_axis_name="c", subcore_axis_name="s"),
        compiler_params=pltpu.CompilerParams(use_tc_tiling_on_sc=False),
        scratch_types=[
            pltpu.VMEM((_NCHUNK, 128), jnp.int32),
            pltpu.VMEM((128, 16), jnp.float32),
            pltpu.VMEM((_TPT, 16), jnp.float32),
            pltpu.VMEM_SHARED((_NPr, 16), jnp.float32),
        ],
    )
    def k(ones_hbm, dsti_hbm, zeros_hbm, out_hbm, didx, vals, tbuf, acc):
        c = lax.axis_index("c")
        s = lax.axis_index("s")
        wid = c * 16 + s
        pltpu.sync_copy(dsti_hbm.at[wid], didx)
        pltpu.sync_copy(ones_hbm, vals)
        pltpu.sync_copy(zeros_hbm, tbuf)
        pltpu.sync_copy(tbuf, acc.at[pl.ds(s * _TPT, _TPT)])
        plsc.subcore_barrier()
        for j in range(_NCHUNK):
            pltpu.sync_copy(vals, acc.at[didx.at[j]], add=True)
        plsc.subcore_barrier()
        pltpu.sync_copy(acc.at[pl.ds(s * _TPT, _TPT)], tbuf)
        pltpu.sync_copy(tbuf, out_hbm.at[c, pl.ds(s * _TPT, _TPT)])

    return k


def _sc_deg(*args):
    return _make_deg()(*args)


def _prep_body(s_ref, d_ref, o_ref):
    s = s_ref[...]
    d = d_ref[...]
    e = (lax.broadcasted_iota(jnp.int32, s.shape, 0) * 128
         + lax.broadcasted_iota(jnp.int32, s.shape, 1))
    o_ref[...] = jnp.where(s == d, N + (e & 15), d)


def _layer0_body(x_ref, w_ref, degp_ref, hs_ref, dis_ref):
    degp = degp_ref[...]
    deg = degp[0, :N, 0:1] + degp[1, :N, 0:1]
    dis = lax.rsqrt(deg + 1.0)
    h = jnp.dot(x_ref[...], w_ref[...], preferred_element_type=jnp.float32)
    hs_ref[...] = dis * h
    dis_ref[...] = dis


def _combine_body(p_ref, hs_ref, dis_ref, b_ref, wn_ref, x_ref, hsn_ref):
    p = p_ref[...]
    s = p[0, :N, :] + p[1, :N, :]
    dis = dis_ref[...]
    xl = jnp.tanh(dis * (s + hs_ref[...]) + b_ref[...])
    x_ref[...] = xl
    hsn_ref[...] = dis * jnp.dot(xl, wn_ref[...],
                                 preferred_element_type=jnp.float32)


def _last_body(p_ref, hs_ref, dis_ref, b_ref, v_ref):
    p = p_ref[...]
    s = p[0, :N, :] + p[1, :N, :]
    t = jnp.tanh(dis_ref[...] * (s + hs_ref[...]) + b_ref[...])
    v_ref[...] = t[:, 0:1]


def _rank_body(v2d_ref, b2d_ref, vs_ref, bs_ref, o_ref):
    big = jnp.int32(2 ** 30)
    allb = b2d_ref[...]
    allflat = (lax.broadcasted_iota(jnp.int32, (80, 128), 0) * 128
               + lax.broadcasted_iota(jnp.int32, (80, 128), 1))
    for blk in range(10):
        vi = v2d_ref[8 * blk:8 * blk + 8, :]
        bi = b2d_ref[8 * blk:8 * blk + 8, :]
        flati = (lax.broadcasted_iota(jnp.int32, (8, 128), 0) * 128
                 + lax.broadcasted_iota(jnp.int32, (8, 128), 1)
                 + 1024 * blk)
        bfirst = bs_ref[8 * blk * 128]
        blast = bs_ref[8 * blk * 128 + 1023]
        span_lo = jnp.min(jnp.where(allb == bfirst, allflat, big))
        span_hi = jnp.max(jnp.where(allb == blast, allflat, -1)) + 1
        lo8 = (span_lo // 8) * 8
        nsteps = (span_hi - lo8 + 7) // 8

        def jbody(t, cnt):
            base = lo8 + t * 8
            for u in range(8):
                j = base + u
                vj = vs_ref[j]
                bj = bs_ref[j]
                beats = (vj > vi) | ((vj == vi) & (j < flati))
                cnt = cnt + jnp.where(beats & (bj == bi), 1, 0)
            return cnt

        cnt = lax.fori_loop(0, nsteps, jbody,
                            jnp.zeros((8, 128), jnp.int32))
        slot = jnp.where((cnt < K) & (bi >= 0), bi * 32 + cnt,
                         3200 + (flati & 127))
        o_ref[8 * blk:8 * blk + 8, :] = slot


def _head1_body(p_ref, w_ref, b_ref, o_ref):
    p = p_ref[...]
    rows = p[0] + p[1]
    o_ref[...] = jax.nn.relu(
        jnp.dot(rows, w_ref[...], preferred_element_type=jnp.float32)
        + b_ref[...])


def _headf_body(y_ref, wall_ref, b6_ref, fw_ref, fb_ref, gw_ref, gb_ref,
                o_ref):
    y = y_ref[...]
    a = jnp.concatenate([y[:, 32 * t:32 * t + 16] for t in range(15)], axis=1)
    b = jnp.concatenate([y[:, 32 * t + 16:32 * t + 32] for t in range(15)],
                        axis=1)
    z = jnp.maximum(a, b)
    zw = jnp.concatenate([z[:, 16 * dt:16 * dt + 176] for dt in range(5)],
                         axis=1)
    y6 = jax.nn.relu(
        jnp.dot(zw, wall_ref[...], preferred_element_type=jnp.float32)
        + b6_ref[...])
    h = jax.nn.relu(
        jnp.dot(y6, fw_ref[...], preferred_element_type=jnp.float32)
        + fb_ref[...])
    logits = jnp.dot(h, gw_ref[...],
                     preferred_element_type=jnp.float32) + gb_ref[...]
    m = jnp.max(logits, axis=1, keepdims=True)
    lse = m + jnp.log(jnp.sum(jnp.exp(logits - m), axis=1, keepdims=True))
    o_ref[...] = logits - lse


def _sds(shape):
    return jax.ShapeDtypeStruct(shape, jnp.float32)


def kernel(x, edge_index, batch, W1, b1, W2, b2, W3, b3, W4, b4, conv5_w,
           conv5_b, conv6_w, conv6_b, fc1_w, fc1_b, fc2_w, fc2_b):
    f32 = jnp.float32
    i32 = jnp.int32
    src = edge_index[0]
    dst = edge_index[1]

    dste2d = pl.pallas_call(
        _prep_body,
        out_shape=jax.ShapeDtypeStruct((2500, 128), i32),
    )(src.reshape(2500, 128), dst.reshape(2500, 128))
    npad = _EPAD - E
    pad_dst = N + (jnp.arange(npad, dtype=i32) & 15)
    pad_src = (jnp.arange(npad, dtype=i32) * 97) % N
    dsti = jnp.concatenate([dste2d.reshape(E), pad_dst]).reshape(32, _NCHUNK, 128)
    srci = jnp.concatenate([src, pad_src]).reshape(32, _NCHUNK, 128)
    z32 = jnp.zeros((_TPT, 32), f32)
    z16 = jnp.zeros((_TPT, 16), f32)

    ones_t = jnp.concatenate([jnp.ones((128, 1), f32), jnp.zeros((128, 15), f32)], 1)
    degp = _sc_deg(ones_t, dsti, z16)

    hs1, dis = pl.pallas_call(
        _layer0_body,
        out_shape=[_sds((N, 32)), _sds((N, 1))],
    )(x, W1, degp)

    def combine(P, hs, b2d, Wn, wout):
        return pl.pallas_call(
            _combine_body,
            out_shape=[_sds((N, 32)), _sds((N, wout))],
        )(P, hs, dis, b2d, Wn)

    P1 = _sc_agg32(hs1, srci, dsti, z32)
    x1, hs2 = combine(P1, hs1, b1.reshape(1, 32), W2, 32)
    P2 = _sc_agg32(hs2, srci, dsti, z32)
    x2, hs3 = combine(P2, hs2, b2.reshape(1, 32), W3, 32)
    P3 = _sc_agg32(hs3, srci, dsti, z32)
    W4p = jnp.pad(W4, ((0, 0), (0, 15)))
    x3, hs4 = combine(P3, hs3, b3.reshape(1, 32), W4p, 16)
    P4 = _sc_agg16(hs4, srci, dsti, z16)
    b4p = jnp.pad(b4.reshape(1, 1), ((0, 0), (0, 15)))
    vcol = pl.pallas_call(
        _last_body,
        out_shape=_sds((N, 1)),
    )(P4, hs4, dis, b4p)

    v2d = jnp.pad(vcol.reshape(N), (0, _NPAD2 - N),
                  constant_values=-jnp.inf).reshape(80, 128)
    batch2d = jnp.pad(batch, (0, _NPAD2 - N),
                      constant_values=-1).reshape(80, 128)
    slot2d = pl.pallas_call(
        _rank_body,
        in_specs=[
            pl.BlockSpec(memory_space=pltpu.VMEM),
            pl.BlockSpec(memory_space=pltpu.VMEM),
            pl.BlockSpec(memory_space=pltpu.SMEM),
            pl.BlockSpec(memory_space=pltpu.SMEM),
        ],
        out_shape=jax.ShapeDtypeStruct((80, 128), i32),
    )(v2d, batch2d, v2d.reshape(_NPAD2), batch2d.reshape(_NPAD2))

    xc = jnp.concatenate([x1, x2, x3, vcol, jnp.zeros((N, 31), f32)], 1)
    xc_ext = jnp.pad(xc, ((0, _NPAD2 - N), (0, 0)))
    npad_n = 32 * 3 * 128 - _NPAD2
    srcn = jnp.concatenate([jnp.arange(_NPAD2, dtype=i32),
                            (jnp.arange(npad_n, dtype=i32) * 97) % N])
    dstn = jnp.concatenate([slot2d.reshape(_NPAD2),
                            3200 + (jnp.arange(npad_n, dtype=i32) & 127)])
    P5 = _sc_slotscatter(xc_ext, srcn.reshape(32, 3, 128),
                         dstn.reshape(32, 3, 128),
                         jnp.zeros((_NSLOT // 16, 128), f32))

    W5p = jnp.pad(conv5_w[:, 0, :].T, ((0, 31), (0, 0)))
    Y5 = pl.pallas_call(
        _head1_body,
        out_shape=_sds((_NSLOT, 16)),
    )(P5, W5p, conv5_b.reshape(1, 16))

    Y5w = Y5[:G * 32].reshape(G, 512)
    eye11 = jnp.eye(11, dtype=f32)
    W_all = jnp.concatenate(
        [jnp.kron(eye11, conv6_w[:, :, dt].T) for dt in range(5)], axis=0)
    b6t = jnp.tile(conv6_b, 11).reshape(1, 352)
    fc1_wr = fc1_w.reshape(32, 11, 128).transpose(1, 0, 2).reshape(352, 128)
    out = pl.pallas_call(
        _headf_body,
        out_shape=_sds((G, 10)),
    )(Y5w, W_all, b6t, fc1_wr, fc1_b.reshape(1, 128),
      fc2_w, fc2_b.reshape(1, 10))
    return out


# speedup vs baseline: 3.7979x; 1.0271x over previous
import functools

import jax
import jax.numpy as jnp
from jax import lax
from jax.experimental import pallas as pl
from jax.experimental.pallas import tpu as pltpu
from jax.experimental.pallas import tpu_sc as plsc

N = 10000
E = 320000
G = 100
K = 30

_NPr = 10112
_TPT = _NPr // 16
_NCHUNK = 80
_EPAD = 32 * _NCHUNK * 128
_NPAD2 = 10240

@functools.lru_cache(maxsize=None)
def _make_edge_agg(W, npr, nchunk):
    tpt = npr // 16

    @functools.partial(
        pl.kernel,
        out_type=jax.ShapeDtypeStruct((2, npr, W), jnp.float32),
        mesh=plsc.VectorSubcoreMesh(core_axis_name="c", subcore_axis_name="s"),
        compiler_params=pltpu.CompilerParams(use_tc_tiling_on_sc=False),
        scratch_types=[
            pltpu.VMEM((nchunk, 128), jnp.int32),
            pltpu.VMEM((nchunk, 128), jnp.int32),
            pltpu.VMEM((128, W), jnp.float32),
            pltpu.VMEM((128, W), jnp.float32),
            pltpu.VMEM((128, W), jnp.float32),
            pltpu.VMEM((128, W), jnp.float32),
            pltpu.VMEM((tpt, W), jnp.float32),
            pltpu.VMEM_SHARED((npr, W), jnp.float32),
            pltpu.SemaphoreType.DMA,
            pltpu.SemaphoreType.DMA,
            pltpu.SemaphoreType.DMA,
            pltpu.SemaphoreType.DMA,
            pltpu.SemaphoreType.DMA,
            pltpu.SemaphoreType.DMA,
            pltpu.SemaphoreType.DMA,
            pltpu.SemaphoreType.DMA,
        ],
    )
    def k(table_hbm, srci_hbm, dsti_hbm, zeros_hbm, out_hbm,
          sidx, didx, r0, r1, r2, r3, tbuf, acc,
          g0, g1, g2, g3, s0, s1, s2, s3):
        c = lax.axis_index("c")
        s = lax.axis_index("s")
        wid = c * 16 + s
        rows = (r0, r1, r2, r3)
        gsem = (g0, g1, g2, g3)
        ssem = (s0, s1, s2, s3)
        pltpu.sync_copy(srci_hbm.at[wid], sidx)
        pltpu.sync_copy(dsti_hbm.at[wid], didx)
        pltpu.sync_copy(zeros_hbm, tbuf)
        pltpu.sync_copy(tbuf, acc.at[pl.ds(s * tpt, tpt)])
        plsc.subcore_barrier()
        pltpu.async_copy(table_hbm.at[sidx.at[0]], rows[0], gsem[0])
        if nchunk > 1:
            pltpu.async_copy(table_hbm.at[sidx.at[1]], rows[1], gsem[1])
        for j in range(nchunk):
            b = j % 4
            pltpu.make_async_copy(table_hbm.at[sidx.at[j]], rows[b],
                                  gsem[b]).wait()
            pltpu.async_copy(rows[b], acc.at[didx.at[j]], ssem[b], add=True)
            if j + 2 < nchunk:
                b2 = (j + 2) % 4
                if j >= 2:
                    pltpu.make_async_copy(rows[b2], acc.at[didx.at[j - 2]],
                                          ssem[b2]).wait()
                pltpu.async_copy(table_hbm.at[sidx.at[j + 2]], rows[b2],
                                 gsem[b2])
        for j in range(max(nchunk - 4, 0), nchunk):
            b = j % 4
            pltpu.make_async_copy(rows[b], acc.at[didx.at[j]],
                                  ssem[b]).wait()
        plsc.subcore_barrier()
        pltpu.sync_copy(acc.at[pl.ds(s * tpt, tpt)], tbuf)
        pltpu.sync_copy(tbuf, out_hbm.at[c, pl.ds(s * tpt, tpt)])

    return k


def _sc_agg32(*args):
    return _make_edge_agg(32, _NPr, _NCHUNK)(*args)


def _sc_agg16(*args):
    return _make_edge_agg(16, _NPr, _NCHUNK)(*args)


_NSLOT = 3328


def _sc_slotscatter(*args):
    return _make_edge_agg(16, _NSLOT, 3)(*args)


@functools.lru_cache(maxsize=None)
def _make_deg():
    @functools.partial(
        pl.kernel,
        out_type=jax.ShapeDtypeStruct((2, _NPr, 16), jnp.float32),
        mesh=plsc.VectorSubcoreMesh(core_axis_name="c", subcore_axis_name="s"),
        compiler_params=pltpu.CompilerParams(use_tc_tiling_on_sc=False),
        scratch_types=[
            pltpu.VMEM((_NCHUNK, 128), jnp.int32),
            pltpu.VMEM((128, 16), jnp.float32),
            pltpu.VMEM((_TPT, 16), jnp.float32),
            pltpu.VMEM_SHARED((_NPr, 16), jnp.float32),
        ],
    )
    def k(ones_hbm, dsti_hbm, zeros_hbm, out_hbm, didx, vals, tbuf, acc):
        c = lax.axis_index("c")
        s = lax.axis_index("s")
        wid = c * 16 + s
        pltpu.sync_copy(dsti_hbm.at[wid], didx)
        pltpu.sync_copy(ones_hbm, vals)
        pltpu.sync_copy(zeros_hbm, tbuf)
        pltpu.sync_copy(tbuf, acc.at[pl.ds(s * _TPT, _TPT)])
        plsc.subcore_barrier()
        for j in range(_NCHUNK):
            pltpu.sync_copy(vals, acc.at[didx.at[j]], add=True)
        plsc.subcore_barrier()
        pltpu.sync_copy(acc.at[pl.ds(s * _TPT, _TPT)], tbuf)
        pltpu.sync_copy(tbuf, out_hbm.at[c, pl.ds(s * _TPT, _TPT)])

    return k


def _sc_deg(*args):
    return _make_deg()(*args)


def _prep_body(s_ref, d_ref, o_ref):
    s = s_ref[...]
    d = d_ref[...]
    e = (lax.broadcasted_iota(jnp.int32, s.shape, 0) * 128
         + lax.broadcasted_iota(jnp.int32, s.shape, 1))
    o_ref[...] = jnp.where(s == d, N + (e & 15), d)


def _layer0_body(x_ref, w_ref, degp_ref, hs_ref, dis_ref):
    degp = degp_ref[...]
    deg = degp[0, :N, 0:1] + degp[1, :N, 0:1]
    dis = lax.rsqrt(deg + 1.0)
    h = jnp.dot(x_ref[...], w_ref[...], preferred_element_type=jnp.float32)
    hs_ref[...] = dis * h
    dis_ref[...] = dis


def _combine_body(p_ref, hs_ref, dis_ref, b_ref, wn_ref, cin_ref, w5_ref,
                  hsn_ref, cout_ref):
    p = p_ref[...]
    s = p[0, :N, :] + p[1, :N, :]
    dis = dis_ref[...]
    xl = jnp.tanh(dis * (s + hs_ref[...]) + b_ref[...])
    hsn_ref[...] = dis * jnp.dot(xl, wn_ref[...],
                                 preferred_element_type=jnp.float32)
    cout_ref[...] = cin_ref[...] + jnp.dot(
        xl, w5_ref[...], preferred_element_type=jnp.float32)


def _last_body(p_ref, hs_ref, dis_ref, b_ref, cin_ref, w5d_ref, b5_ref,
               rb5_ref, v_ref, q_ref):
    p = p_ref[...]
    s = p[0, :N, :] + p[1, :N, :]
    t = jnp.tanh(dis_ref[...] * (s + hs_ref[...]) + b_ref[...])
    v = t[:, 0:1]
    v_ref[...] = v
    q_ref[...] = jax.nn.relu(cin_ref[...] + v * w5d_ref[...]
                             + b5_ref[...]) - rb5_ref[...]


def _rank_body(v2d_ref, b2d_ref, vs_ref, bs_ref, o_ref):
    big = jnp.int32(2 ** 30)
    allb = b2d_ref[...]
    allflat = (lax.broadcasted_iota(jnp.int32, (80, 128), 0) * 128
               + lax.broadcasted_iota(jnp.int32, (80, 128), 1))
    for blk in range(10):
        vi = v2d_ref[8 * blk:8 * blk + 8, :]
        bi = b2d_ref[8 * blk:8 * blk + 8, :]
        flati = (lax.broadcasted_iota(jnp.int32, (8, 128), 0) * 128
                 + lax.broadcasted_iota(jnp.int32, (8, 128), 1)
                 + 1024 * blk)
        bfirst = bs_ref[8 * blk * 128]
        blast = bs_ref[8 * blk * 128 + 1023]
        span_lo = jnp.min(jnp.where(allb == bfirst, allflat, big))
        span_hi = jnp.max(jnp.where(allb == blast, allflat, -1)) + 1
        lo8 = (span_lo // 8) * 8
        nsteps = (span_hi - lo8 + 7) // 8

        def jbody(t, cnt):
            base = lo8 + t * 8
            for u in range(8):
                j = base + u
                vj = vs_ref[j]
                bj = bs_ref[j]
                beats = (vj > vi) | ((vj == vi) & (j < flati))
                cnt = cnt + jnp.where(beats & (bj == bi), 1, 0)
            return cnt

        cnt = lax.fori_loop(0, nsteps, jbody,
                            jnp.zeros((8, 128), jnp.int32))
        slot = jnp.where((cnt < K) & (bi >= 0), bi * 32 + cnt,
                         3200 + (flati & 127))
        o_ref[8 * blk:8 * blk + 8, :] = slot


def _head0_body(p_ref, rb5_ref, o_ref):
    p = p_ref[...]
    o_ref[...] = p[0] + p[1] + rb5_ref[...]


def _headf_body(y_ref, wall_ref, b6_ref, fw_ref, fb_ref, gw_ref, gb_ref,
                o_ref):
    y = y_ref[...]
    a = jnp.concatenate([y[:, 32 * t:32 * t + 16] for t in range(15)], axis=1)
    b = jnp.concatenate([y[:, 32 * t + 16:32 * t + 32] for t in range(15)],
                        axis=1)
    z = jnp.maximum(a, b)
    zw = jnp.concatenate([z[:, 16 * dt:16 * dt + 176] for dt in range(5)],
                         axis=1)
    y6 = jax.nn.relu(
        jnp.dot(zw, wall_ref[...], preferred_element_type=jnp.float32)
        + b6_ref[...])
    h = jax.nn.relu(
        jnp.dot(y6, fw_ref[...], preferred_element_type=jnp.float32)
        + fb_ref[...])
    logits = jnp.dot(h, gw_ref[...],
                     preferred_element_type=jnp.float32) + gb_ref[...]
    m = jnp.max(logits, axis=1, keepdims=True)
    lse = m + jnp.log(jnp.sum(jnp.exp(logits - m), axis=1, keepdims=True))
    o_ref[...] = logits - lse


def _sds(shape):
    return jax.ShapeDtypeStruct(shape, jnp.float32)


def kernel(x, edge_index, batch, W1, b1, W2, b2, W3, b3, W4, b4, conv5_w,
           conv5_b, conv6_w, conv6_b, fc1_w, fc1_b, fc2_w, fc2_b):
    f32 = jnp.float32
    i32 = jnp.int32
    src = edge_index[0]
    dst = edge_index[1]

    dste2d = pl.pallas_call(
        _prep_body,
        out_shape=jax.ShapeDtypeStruct((2500, 128), i32),
    )(src.reshape(2500, 128), dst.reshape(2500, 128))
    npad = _EPAD - E
    pad_dst = N + (jnp.arange(npad, dtype=i32) & 15)
    pad_src = (jnp.arange(npad, dtype=i32) * 97) % N
    dsti = jnp.concatenate([dste2d.reshape(E), pad_dst]).reshape(32, _NCHUNK, 128)
    srci = jnp.concatenate([src, pad_src]).reshape(32, _NCHUNK, 128)
    z32 = jnp.zeros((_TPT, 32), f32)
    z16 = jnp.zeros((_TPT, 16), f32)

    ones_t = jnp.concatenate([jnp.ones((128, 1), f32), jnp.zeros((128, 15), f32)], 1)
    degp = _sc_deg(ones_t, dsti, z16)

    hs1, dis = pl.pallas_call(
        _layer0_body,
        out_shape=[_sds((N, 32)), _sds((N, 1))],
    )(x, W1, degp)

    W5T = conv5_w[:, 0, :].T
    b5row = conv5_b.reshape(1, 16)
    rb5 = jax.nn.relu(b5row)

    def combine(P, hs, b2d, Wn, wout, cin, w5part):
        return pl.pallas_call(
            _combine_body,
            out_shape=[_sds((N, wout)), _sds((N, 16))],
        )(P, hs, dis, b2d, Wn, cin, w5part)

    P1 = _sc_agg32(hs1, srci, dsti, z32)
    hs2, c1 = combine(P1, hs1, b1.reshape(1, 32), W2, 32,
                      jnp.zeros((N, 16), f32), W5T[0:32])
    P2 = _sc_agg32(hs2, srci, dsti, z32)
    hs3, c2 = combine(P2, hs2, b2.reshape(1, 32), W3, 32, c1, W5T[32:64])
    P3 = _sc_agg32(hs3, srci, dsti, z32)
    W4p = jnp.pad(W4, ((0, 0), (0, 15)))
    hs4, c3 = combine(P3, hs3, b3.reshape(1, 32), W4p, 16, c2, W5T[64:96])
    P4 = _sc_agg16(hs4, srci, dsti, z16)
    b4p = jnp.pad(b4.reshape(1, 1), ((0, 0), (0, 15)))
    vcol, Q = pl.pallas_call(
        _last_body,
        out_shape=[_sds((N, 1)), _sds((N, 16))],
    )(P4, hs4, dis, b4p, c3, W5T[96:97], b5row, rb5)

    v2d = jnp.pad(vcol.reshape(N), (0, _NPAD2 - N),
                  constant_values=-jnp.inf).reshape(80, 128)
    batch2d = jnp.pad(batch, (0, _NPAD2 - N),
                      constant_values=-1).reshape(80, 128)
    slot2d = pl.pallas_call(
        _rank_body,
        in_specs=[
            pl.BlockSpec(memory_space=pltpu.VMEM),
            pl.BlockSpec(memory_space=pltpu.VMEM),
            pl.BlockSpec(memory_space=pltpu.SMEM),
            pl.BlockSpec(memory_space=pltpu.SMEM),
        ],
        out_shape=jax.ShapeDtypeStruct((80, 128), i32),
    )(v2d, batch2d, v2d.reshape(_NPAD2), batch2d.reshape(_NPAD2))

    q_ext = jnp.pad(Q, ((0, _NPAD2 - N), (0, 0)))
    npad_n = 32 * 3 * 128 - _NPAD2
    srcn = jnp.concatenate([jnp.arange(_NPAD2, dtype=i32),
                            (jnp.arange(npad_n, dtype=i32) * 97) % N])
    dstn = jnp.concatenate([slot2d.reshape(_NPAD2),
                            3200 + (jnp.arange(npad_n, dtype=i32) & 127)])
    P5 = _sc_slotscatter(q_ext, srcn.reshape(32, 3, 128),
                         dstn.reshape(32, 3, 128),
                         jnp.zeros((_NSLOT // 16, 16), f32))

    Y5 = pl.pallas_call(
        _head0_body,
        out_shape=_sds((_NSLOT, 16)),
    )(P5, rb5)

    Y5w = Y5[:G * 32].reshape(G, 512)
    eye11 = jnp.eye(11, dtype=f32)
    W_all = jnp.concatenate(
        [jnp.kron(eye11, conv6_w[:, :, dt].T) for dt in range(5)], axis=0)
    b6t = jnp.tile(conv6_b, 11).reshape(1, 352)
    fc1_wr = fc1_w.reshape(32, 11, 128).transpose(1, 0, 2).reshape(352, 128)
    out = pl.pallas_call(
        _headf_body,
        out_shape=_sds((G, 10)),
    )(Y5w, W_all, b6t, fc1_wr, fc1_b.reshape(1, 128),
      fc2_w, fc2_b.reshape(1, 10))
    return out


# speedup vs baseline: 4.1606x; 1.0955x over previous
import functools

import jax
import jax.numpy as jnp
from jax import lax
from jax.experimental import pallas as pl
from jax.experimental.pallas import tpu as pltpu
from jax.experimental.pallas import tpu_sc as plsc

N = 10000
E = 320000
G = 100
K = 30

_NPr = 10112
_TPT = _NPr // 16
_NCHUNK = 80
_EPAD = 32 * _NCHUNK * 128
_NPAD2 = 10240

@functools.lru_cache(maxsize=None)
def _make_edge_agg(W, npr, nchunk, trows):
    tpt = npr // 16
    tr16 = trows // 16

    @functools.partial(
        pl.kernel,
        out_type=jax.ShapeDtypeStruct((2, npr, W), jnp.float32),
        mesh=plsc.VectorSubcoreMesh(core_axis_name="c", subcore_axis_name="s"),
        compiler_params=pltpu.CompilerParams(use_tc_tiling_on_sc=False),
        scratch_types=[
            pltpu.VMEM((nchunk, 128), jnp.int32),
            pltpu.VMEM((nchunk, 128), jnp.int32),
            pltpu.VMEM((128, W), jnp.float32),
            pltpu.VMEM((128, W), jnp.float32),
            pltpu.VMEM((128, W), jnp.float32),
            pltpu.VMEM((128, W), jnp.float32),
            pltpu.VMEM((tpt, W), jnp.float32),
            pltpu.VMEM((tr16, W), jnp.float32),
            pltpu.VMEM_SHARED((npr, W), jnp.float32),
            pltpu.VMEM_SHARED((trows, W), jnp.float32),
            pltpu.SemaphoreType.DMA,
            pltpu.SemaphoreType.DMA,
            pltpu.SemaphoreType.DMA,
            pltpu.SemaphoreType.DMA,
            pltpu.SemaphoreType.DMA,
            pltpu.SemaphoreType.DMA,
            pltpu.SemaphoreType.DMA,
            pltpu.SemaphoreType.DMA,
        ],
    )
    def k(table_hbm, srci_hbm, dsti_hbm, zeros_hbm, out_hbm,
          sidx, didx, r0, r1, r2, r3, tbuf, tstage, acc, tabs,
          g0, g1, g2, g3, s0, s1, s2, s3):
        c = lax.axis_index("c")
        s = lax.axis_index("s")
        wid = c * 16 + s
        rows = (r0, r1, r2, r3)
        gsem = (g0, g1, g2, g3)
        ssem = (s0, s1, s2, s3)
        pltpu.sync_copy(srci_hbm.at[wid], sidx)
        pltpu.sync_copy(dsti_hbm.at[wid], didx)
        pltpu.sync_copy(zeros_hbm, tbuf)
        pltpu.sync_copy(tbuf, acc.at[pl.ds(s * tpt, tpt)])
        pltpu.sync_copy(table_hbm.at[pl.ds(s * tr16, tr16)], tstage)
        pltpu.sync_copy(tstage, tabs.at[pl.ds(s * tr16, tr16)])
        plsc.subcore_barrier()
        pltpu.async_copy(tabs.at[sidx.at[0]], rows[0], gsem[0])
        if nchunk > 1:
            pltpu.async_copy(tabs.at[sidx.at[1]], rows[1], gsem[1])
        for j in range(nchunk):
            b = j % 4
            pltpu.make_async_copy(tabs.at[sidx.at[j]], rows[b],
                                  gsem[b]).wait()
            pltpu.async_copy(rows[b], acc.at[didx.at[j]], ssem[b], add=True)
            if j + 2 < nchunk:
                b2 = (j + 2) % 4
                if j >= 2:
                    pltpu.make_async_copy(rows[b2], acc.at[didx.at[j - 2]],
                                          ssem[b2]).wait()
                pltpu.async_copy(tabs.at[sidx.at[j + 2]], rows[b2],
                                 gsem[b2])
        for j in range(max(nchunk - 4, 0), nchunk):
            b = j % 4
            pltpu.make_async_copy(rows[b], acc.at[didx.at[j]],
                                  ssem[b]).wait()
        plsc.subcore_barrier()
        pltpu.sync_copy(acc.at[pl.ds(s * tpt, tpt)], tbuf)
        pltpu.sync_copy(tbuf, out_hbm.at[c, pl.ds(s * tpt, tpt)])

    return k


def _sc_agg32(table, *args):
    table = jnp.pad(table, ((0, _NPr - N), (0, 0)))
    return _make_edge_agg(32, _NPr, _NCHUNK, _NPr)(table, *args)


def _sc_agg16(table, *args):
    table = jnp.pad(table, ((0, _NPr - N), (0, 0)))
    return _make_edge_agg(16, _NPr, _NCHUNK, _NPr)(table, *args)


_NSLOT = 3328


def _sc_slotscatter(*args):
    return _make_edge_agg(16, _NSLOT, 3, _NPAD2)(*args)


@functools.lru_cache(maxsize=None)
def _make_deg():
    @functools.partial(
        pl.kernel,
        out_type=jax.ShapeDtypeStruct((2, _NPr, 16), jnp.float32),
        mesh=plsc.VectorSubcoreMesh(core_axis_name="c", subcore_axis_name="s"),
        compiler_params=pltpu.CompilerParams(use_tc_tiling_on_sc=False),
        scratch_types=[
            pltpu.VMEM((_NCHUNK, 128), jnp.int32),
            pltpu.VMEM((128, 16), jnp.float32),
            pltpu.VMEM((_TPT, 16), jnp.float32),
            pltpu.VMEM_SHARED((_NPr, 16), jnp.float32),
        ],
    )
    def k(ones_hbm, dsti_hbm, zeros_hbm, out_hbm, didx, vals, tbuf, acc):
        c = lax.axis_index("c")
        s = lax.axis_index("s")
        wid = c * 16 + s
        pltpu.sync_copy(dsti_hbm.at[wid], didx)
        pltpu.sync_copy(ones_hbm, vals)
        pltpu.sync_copy(zeros_hbm, tbuf)
        pltpu.sync_copy(tbuf, acc.at[pl.ds(s * _TPT, _TPT)])
        plsc.subcore_barrier()
        for j in range(_NCHUNK):
            pltpu.sync_copy(vals, acc.at[didx.at[j]], add=True)
        plsc.subcore_barrier()
        pltpu.sync_copy(acc.at[pl.ds(s * _TPT, _TPT)], tbuf)
        pltpu.sync_copy(tbuf, out_hbm.at[c, pl.ds(s * _TPT, _TPT)])

    return k


def _sc_deg(*args):
    return _make_deg()(*args)


def _prep_body(s_ref, d_ref, o_ref):
    s = s_ref[...]
    d = d_ref[...]
    e = (lax.broadcasted_iota(jnp.int32, s.shape, 0) * 128
         + lax.broadcasted_iota(jnp.int32, s.shape, 1))
    o_ref[...] = jnp.where(s == d, N + (e & 15), d)


def _layer0_body(x_ref, w_ref, degp_ref, hs_ref, dis_ref):
    degp = degp_ref[...]
    deg = degp[0, :N, 0:1] + degp[1, :N, 0:1]
    dis = lax.rsqrt(deg + 1.0)
    h = jnp.dot(x_ref[...], w_ref[...], preferred_element_type=jnp.float32)
    hs_ref[...] = dis * h
    dis_ref[...] = dis


def _combine_body(p_ref, hs_ref, dis_ref, b_ref, wn_ref, cin_ref, w5_ref,
                  hsn_ref, cout_ref):
    p = p_ref[...]
    s = p[0, :N, :] + p[1, :N, :]
    dis = dis_ref[...]
    xl = jnp.tanh(dis * (s + hs_ref[...]) + b_ref[...])
    hsn_ref[...] = dis * jnp.dot(xl, wn_ref[...],
                                 preferred_element_type=jnp.float32)
    cout_ref[...] = cin_ref[...] + jnp.dot(
        xl, w5_ref[...], preferred_element_type=jnp.float32)


def _last_body(p_ref, hs_ref, dis_ref, b_ref, cin_ref, w5d_ref, b5_ref,
               rb5_ref, v_ref, q_ref):
    p = p_ref[...]
    s = p[0, :N, :] + p[1, :N, :]
    t = jnp.tanh(dis_ref[...] * (s + hs_ref[...]) + b_ref[...])
    v = t[:, 0:1]
    v_ref[...] = v
    q_ref[...] = jax.nn.relu(cin_ref[...] + v * w5d_ref[...]
                             + b5_ref[...]) - rb5_ref[...]


def _rank_body(v2d_ref, b2d_ref, vs_ref, bs_ref, o_ref):
    big = jnp.int32(2 ** 30)
    allb = b2d_ref[...]
    allflat = (lax.broadcasted_iota(jnp.int32, (80, 128), 0) * 128
               + lax.broadcasted_iota(jnp.int32, (80, 128), 1))
    for blk in range(10):
        vi = v2d_ref[8 * blk:8 * blk + 8, :]
        bi = b2d_ref[8 * blk:8 * blk + 8, :]
        flati = (lax.broadcasted_iota(jnp.int32, (8, 128), 0) * 128
                 + lax.broadcasted_iota(jnp.int32, (8, 128), 1)
                 + 1024 * blk)
        bfirst = bs_ref[8 * blk * 128]
        blast = bs_ref[8 * blk * 128 + 1023]
        span_lo = jnp.min(jnp.where(allb == bfirst, allflat, big))
        span_hi = jnp.max(jnp.where(allb == blast, allflat, -1)) + 1
        lo8 = (span_lo // 8) * 8
        nsteps = (span_hi - lo8 + 7) // 8

        def jbody(t, cnt):
            base = lo8 + t * 8
            for u in range(8):
                j = base + u
                vj = vs_ref[j]
                bj = bs_ref[j]
                beats = (vj > vi) | ((vj == vi) & (j < flati))
                cnt = cnt + jnp.where(beats & (bj == bi), 1, 0)
            return cnt

        cnt = lax.fori_loop(0, nsteps, jbody,
                            jnp.zeros((8, 128), jnp.int32))
        slot = jnp.where((cnt < K) & (bi >= 0), bi * 32 + cnt,
                         3200 + (flati & 127))
        o_ref[8 * blk:8 * blk + 8, :] = slot


def _head0_body(p_ref, rb5_ref, o_ref):
    p = p_ref[...]
    o_ref[...] = p[0] + p[1] + rb5_ref[...]


def _headf_body(y_ref, wall_ref, b6_ref, fw_ref, fb_ref, gw_ref, gb_ref,
                o_ref):
    y = y_ref[...]
    a = jnp.concatenate([y[:, 32 * t:32 * t + 16] for t in range(15)], axis=1)
    b = jnp.concatenate([y[:, 32 * t + 16:32 * t + 32] for t in range(15)],
                        axis=1)
    z = jnp.maximum(a, b)
    zw = jnp.concatenate([z[:, 16 * dt:16 * dt + 176] for dt in range(5)],
                         axis=1)
    y6 = jax.nn.relu(
        jnp.dot(zw, wall_ref[...], preferred_element_type=jnp.float32)
        + b6_ref[...])
    h = jax.nn.relu(
        jnp.dot(y6, fw_ref[...], preferred_element_type=jnp.float32)
        + fb_ref[...])
    logits = jnp.dot(h, gw_ref[...],
                     preferred_element_type=jnp.float32) + gb_ref[...]
    m = jnp.max(logits, axis=1, keepdims=True)
    lse = m + jnp.log(jnp.sum(jnp.exp(logits - m), axis=1, keepdims=True))
    o_ref[...] = logits - lse


def _sds(shape):
    return jax.ShapeDtypeStruct(shape, jnp.float32)


def kernel(x, edge_index, batch, W1, b1, W2, b2, W3, b3, W4, b4, conv5_w,
           conv5_b, conv6_w, conv6_b, fc1_w, fc1_b, fc2_w, fc2_b):
    f32 = jnp.float32
    i32 = jnp.int32
    src = edge_index[0]
    dst = edge_index[1]

    dste2d = pl.pallas_call(
        _prep_body,
        out_shape=jax.ShapeDtypeStruct((2500, 128), i32),
    )(src.reshape(2500, 128), dst.reshape(2500, 128))
    npad = _EPAD - E
    pad_dst = N + (jnp.arange(npad, dtype=i32) & 15)
    pad_src = (jnp.arange(npad, dtype=i32) * 97) % N
    dsti = jnp.concatenate([dste2d.reshape(E), pad_dst]).reshape(32, _NCHUNK, 128)
    srci = jnp.concatenate([src, pad_src]).reshape(32, _NCHUNK, 128)
    z32 = jnp.zeros((_TPT, 32), f32)
    z16 = jnp.zeros((_TPT, 16), f32)

    ones_t = jnp.concatenate([jnp.ones((128, 1), f32), jnp.zeros((128, 15), f32)], 1)
    degp = _sc_deg(ones_t, dsti, z16)

    hs1, dis = pl.pallas_call(
        _layer0_body,
        out_shape=[_sds((N, 32)), _sds((N, 1))],
    )(x, W1, degp)

    W5T = conv5_w[:, 0, :].T
    b5row = conv5_b.reshape(1, 16)
    rb5 = jax.nn.relu(b5row)

    def combine(P, hs, b2d, Wn, wout, cin, w5part):
        return pl.pallas_call(
            _combine_body,
            out_shape=[_sds((N, wout)), _sds((N, 16))],
        )(P, hs, dis, b2d, Wn, cin, w5part)

    P1 = _sc_agg32(hs1, srci, dsti, z32)
    hs2, c1 = combine(P1, hs1, b1.reshape(1, 32), W2, 32,
                      jnp.zeros((N, 16), f32), W5T[0:32])
    P2 = _sc_agg32(hs2, srci, dsti, z32)
    hs3, c2 = combine(P2, hs2, b2.reshape(1, 32), W3, 32, c1, W5T[32:64])
    P3 = _sc_agg32(hs3, srci, dsti, z32)
    W4p = jnp.pad(W4, ((0, 0), (0, 15)))
    hs4, c3 = combine(P3, hs3, b3.reshape(1, 32), W4p, 16, c2, W5T[64:96])
    P4 = _sc_agg16(hs4, srci, dsti, z16)
    b4p = jnp.pad(b4.reshape(1, 1), ((0, 0), (0, 15)))
    vcol, Q = pl.pallas_call(
        _last_body,
        out_shape=[_sds((N, 1)), _sds((N, 16))],
    )(P4, hs4, dis, b4p, c3, W5T[96:97], b5row, rb5)

    v2d = jnp.pad(vcol.reshape(N), (0, _NPAD2 - N),
                  constant_values=-jnp.inf).reshape(80, 128)
    batch2d = jnp.pad(batch, (0, _NPAD2 - N),
                      constant_values=-1).reshape(80, 128)
    slot2d = pl.pallas_call(
        _rank_body,
        in_specs=[
            pl.BlockSpec(memory_space=pltpu.VMEM),
            pl.BlockSpec(memory_space=pltpu.VMEM),
            pl.BlockSpec(memory_space=pltpu.SMEM),
            pl.BlockSpec(memory_space=pltpu.SMEM),
        ],
        out_shape=jax.ShapeDtypeStruct((80, 128), i32),
    )(v2d, batch2d, v2d.reshape(_NPAD2), batch2d.reshape(_NPAD2))

    q_ext = jnp.pad(Q, ((0, _NPAD2 - N), (0, 0)))
    npad_n = 32 * 3 * 128 - _NPAD2
    srcn = jnp.concatenate([jnp.arange(_NPAD2, dtype=i32),
                            (jnp.arange(npad_n, dtype=i32) * 97) % N])
    dstn = jnp.concatenate([slot2d.reshape(_NPAD2),
                            3200 + (jnp.arange(npad_n, dtype=i32) & 127)])
    P5 = _sc_slotscatter(q_ext, srcn.reshape(32, 3, 128),
                         dstn.reshape(32, 3, 128),
                         jnp.zeros((_NSLOT // 16, 16), f32))

    Y5 = pl.pallas_call(
        _head0_body,
        out_shape=_sds((_NSLOT, 16)),
    )(P5, rb5)

    Y5w = Y5[:G * 32].reshape(G, 512)
    eye11 = jnp.eye(11, dtype=f32)
    W_all = jnp.concatenate(
        [jnp.kron(eye11, conv6_w[:, :, dt].T) for dt in range(5)], axis=0)
    b6t = jnp.tile(conv6_b, 11).reshape(1, 352)
    fc1_wr = fc1_w.reshape(32, 11, 128).transpose(1, 0, 2).reshape(352, 128)
    out = pl.pallas_call(
        _headf_body,
        out_shape=_sds((G, 10)),
    )(Y5w, W_all, b6t, fc1_wr, fc1_b.reshape(1, 128),
      fc2_w, fc2_b.reshape(1, 10))
    return out
